# baseline (speedup 1.0000x reference)
import functools

import jax
import jax.numpy as jnp
import numpy as np
from jax import lax
from jax.experimental import pallas as pl
from jax.experimental.pallas import tpu as pltpu
from jax.experimental.pallas import tpu_sc as plsc

M_ = 8
L_ = 4
P_ = 4
D_ = 32
C_ = 256
LQ_ = 4096
LEN_IN_ = 43520
NROWS_ = LEN_IN_ * M_

_LVL_T = np.array([8, 8, 8, 8], dtype=np.int64)
_LVL_H = np.array([64, 32, 16, 8], dtype=np.int64)
_LVL_W = np.array([64, 32, 16, 8], dtype=np.int64)
_LVL_START = np.array([0, 32768, 40960, 43008], dtype=np.int64)

_J_L = np.array([(j % 16) // 4 for j in range(128)])
_J_M = np.array([j // 16 for j in range(128)])
_LANE_W_F = _LVL_W[_J_L].astype(np.float32)[None, :]
_LANE_H_F = _LVL_H[_J_L].astype(np.float32)[None, :]
_LANE_T_F = _LVL_T[_J_L].astype(np.float32)[None, :]
_LANE_W_I = _LVL_W[_J_L].astype(np.int32)[None, :]
_LANE_H_I = _LVL_H[_J_L].astype(np.int32)[None, :]
_LANE_T_I = _LVL_T[_J_L].astype(np.int32)[None, :]
_LANE_START_I = _LVL_START[_J_L].astype(np.int32)[None, :]
_LANE_M_I = _J_M.astype(np.int32)[None, :]

_BQ = 512
_BV = 512


def _matbias_body(x_ref, w_ref, b_ref, o_ref):
    o_ref[...] = (
        jnp.dot(x_ref[...], w_ref[...], preferred_element_type=jnp.float32)
        + b_ref[...]
    ).astype(o_ref.dtype)


def _matbias(x, w, b, bm, out_dtype=jnp.float32):
    n, k = x.shape
    ko = w.shape[1]
    return pl.pallas_call(
        _matbias_body,
        grid=(n // bm,),
        in_specs=[
            pl.BlockSpec((bm, k), lambda i: (i, 0)),
            pl.BlockSpec((k, ko), lambda i: (0, 0)),
            pl.BlockSpec((1, ko), lambda i: (0, 0)),
        ],
        out_specs=pl.BlockSpec((bm, ko), lambda i: (i, 0)),
        out_shape=jax.ShapeDtypeStruct((n, ko), out_dtype),
    )(x, w, b.reshape(1, ko))


def _value_proj_body(x_ref, w_ref, b_ref, o_ref):
    r = (
        jnp.dot(x_ref[...], w_ref[...], preferred_element_type=jnp.float32)
        + b_ref[...]
    ).astype(jnp.bfloat16)
    for m in range(M_):
        o_ref[:, m, :] = r[:, m * D_:(m + 1) * D_]


def _value_proj(x, w, b):
    n, k = x.shape
    return pl.pallas_call(
        _value_proj_body,
        grid=(n // _BV,),
        in_specs=[
            pl.BlockSpec((_BV, k), lambda i: (i, 0)),
            pl.BlockSpec((k, C_), lambda i: (0, 0)),
            pl.BlockSpec((1, C_), lambda i: (0, 0)),
        ],
        out_specs=pl.BlockSpec((_BV, M_, D_), lambda i: (i, 0, 0)),
        out_shape=jax.ShapeDtypeStruct((n, M_, D_), jnp.bfloat16),
    )(x, w, b.reshape(1, C_))


def _sample_prep_body(q_ref, woff_ref, boff_ref, wattn_ref, battn_ref, rp_ref,
                      idx_ref, w_ref):
    q = q_ref[...]
    off = (
        jnp.dot(q, woff_ref[...], preferred_element_type=jnp.float32)
        + boff_ref[...]
    )
    logits = (
        jnp.dot(q, wattn_ref[...], preferred_element_type=jnp.float32)
        + battn_ref[...]
    )
    a3 = logits.reshape(_BQ, M_, L_ * P_)
    a3 = a3 - jnp.max(a3, axis=-1, keepdims=True)
    e3 = jnp.exp(a3)
    aw = (e3 / jnp.sum(e3, axis=-1, keepdims=True)).reshape(_BQ, 128)

    j = lax.broadcasted_iota(jnp.int32, (1, 128), 1)
    m_i = j // 16
    lvl = (j % 16) // 4
    wi = jnp.right_shift(64, lvl)
    hi = wi
    ti = 8
    start_i = jnp.where(
        lvl == 0, 0,
        jnp.where(lvl == 1, 32768, jnp.where(lvl == 2, 40960, 43008)))
    wf = wi.astype(jnp.float32)
    hf = wf
    tf = 8.0

    ix = (rp_ref[:, 0, :] + off[:, 0:128] / tf) * wf - 0.5
    iy = (rp_ref[:, 1, :] + off[:, 128:256] / wf) * hf - 0.5
    iz = (rp_ref[:, 2, :] + off[:, 256:384] / hf) * tf - 0.5

    x0f = jnp.floor(ix)
    y0f = jnp.floor(iy)
    z0f = jnp.floor(iz)
    fx = ix - x0f
    fy = iy - y0f
    fz = iz - z0f
    x0 = x0f.astype(jnp.int32)
    y0 = y0f.astype(jnp.int32)
    z0 = z0f.astype(jnp.int32)

    vx = []
    vy = []
    vz = []
    xcol = []
    yrow = []
    zrow = []
    base = start_i * M_ + m_i
    for d in (0, 1):
        xi = x0 + d
        yi = y0 + d
        zi = z0 + d
        vx.append((xi >= 0) & (xi < wi))
        vy.append((yi >= 0) & (yi < hi))
        vz.append((zi >= 0) & (zi < ti))
        xcol.append(jnp.clip(xi, 0, wi - 1) * M_)
        yrow.append(jnp.clip(yi, 0, hi - 1) * (wi * M_))
        zrow.append(jnp.clip(zi, 0, ti - 1) * hi * (wi * M_) + base)
    wxa = (1.0 - fx, fx)
    wya = (1.0 - fy, fy)
    wza = (aw * (1.0 - fz), aw * fz)
    for dy in (0, 1):
        for dx in (0, 1):
            vxy = vx[dx] & vy[dy]
            wxy = wxa[dx] * wya[dy]
            sxy = yrow[dy] + xcol[dx]
            for dz in (0, 1):
                c = dz * 4 + dy * 2 + dx
                row = zrow[dz] + sxy
                wgt = jnp.where(vxy & vz[dz], wxy * wza[dz], 0.0)
                idx_ref[:, c, :] = row
                w_ref[:, c, :] = wgt


def _sample_prep(query, woff_r, boff_r, wattn, battn, rp_lane):
    return pl.pallas_call(
        _sample_prep_body,
        grid=(LQ_ // _BQ,),
        in_specs=[
            pl.BlockSpec((_BQ, C_), lambda i: (i, 0)),
            pl.BlockSpec((C_, 3 * 128), lambda i: (0, 0)),
            pl.BlockSpec((1, 3 * 128), lambda i: (0, 0)),
            pl.BlockSpec((C_, 128), lambda i: (0, 0)),
            pl.BlockSpec((1, 128), lambda i: (0, 0)),
            pl.BlockSpec((_BQ, 3, 128), lambda i: (i, 0, 0)),
        ],
        out_specs=[
            pl.BlockSpec((_BQ, 8, 128), lambda i: (i, 0, 0)),
            pl.BlockSpec((_BQ, 8, 128), lambda i: (i, 0, 0)),
        ],
        out_shape=[
            jax.ShapeDtypeStruct((LQ_, 8, 128), jnp.int32),
            jax.ShapeDtypeStruct((LQ_, 8, 128), jnp.float32),
        ],
    )(query, woff_r, boff_r, wattn, battn, rp_lane)


def _sc_sample(table, idxs, wgts):
    info = plsc.get_sparse_core_info()
    nc, ns = info.num_cores, info.num_subcores
    nw = nc * ns
    qpw = LQ_ // nw
    mesh = plsc.VectorSubcoreMesh(core_axis_name="c", subcore_axis_name="s")

    @functools.partial(
        pl.kernel,
        out_type=jax.ShapeDtypeStruct((LQ_, M_, D_), jnp.float32),
        mesh=mesh,
        compiler_params=pltpu.CompilerParams(
            use_tc_tiling_on_sc=False, needs_layout_passes=False),
        scratch_types=[
            pltpu.VMEM((2, 8, 128), jnp.int32),
            pltpu.VMEM((2, 8, 128), jnp.float32),
            pltpu.VMEM((2, 8, 128, D_), jnp.bfloat16),
            pltpu.VMEM((2, M_, D_), jnp.float32),
            pltpu.SemaphoreType.DMA,
            pltpu.SemaphoreType.DMA,
            pltpu.SemaphoreType.DMA,
            pltpu.SemaphoreType.DMA,
            pltpu.SemaphoreType.DMA,
            pltpu.SemaphoreType.DMA,
            pltpu.SemaphoreType.DMA,
            pltpu.SemaphoreType.DMA,
        ],
    )
    def k(table_hbm, idx_hbm, w_hbm, out_hbm, idx_v, w_v, rows_v, out_v,
          si0, si1, sw0, sw1, sg0, sg1, so0, so1):
        wid = lax.axis_index("s") * nc + lax.axis_index("c")
        qbase = wid * qpw
        sem_i = (si0, si1)
        sem_w = (sw0, sw1)
        sem_g = (sg0, sg1)
        sem_o = (so0, so1)

        def load_idx(g, b):
            pltpu.async_copy(idx_hbm.at[g], idx_v.at[b], sem_i[b])

        def wait_idx(g, b):
            pltpu.make_async_copy(idx_hbm.at[g], idx_v.at[b], sem_i[b]).wait()

        def load_w(g, b):
            pltpu.async_copy(w_hbm.at[g], w_v.at[b], sem_w[b])

        def wait_w(g, b):
            pltpu.make_async_copy(w_hbm.at[g], w_v.at[b], sem_w[b]).wait()

        def fire_gathers(b):
            for c in range(8):
                pltpu.async_copy(
                    table_hbm.at[idx_v.at[b, c]], rows_v.at[b, c], sem_g[b])

        def wait_gathers(b):
            pltpu.make_async_copy(
                table_hbm.at[idx_v.at[b, 0]], rows_v.at[b], sem_g[b]).wait()

        def wait_out(b):
            pltpu.make_async_copy(
                out_v.at[b], out_hbm.at[qbase], sem_o[b]).wait()

        def compute_store(g, b):
            def cbody(ci, accs):
                accs = list(accs)
                for c in (ci * 2, ci * 2 + 1):
                  for m in range(M_):
                    a0 = accs[2 * m]
                    a1 = accs[2 * m + 1]
                    wvec = w_v[b, c, pl.ds(m * 16, 16)]
                    for kk in range(16):
                        j = m * 16 + kk
                        wv = wvec[kk]
                        lo, hi = plsc.unpack(
                            rows_v[b, c, j, 0:32],
                            format=plsc.PackFormat.INTERLEAVED)
                        a0 = a0 + wv * lo
                        a1 = a1 + wv * hi
                    accs[2 * m] = a0
                    accs[2 * m + 1] = a1
                return tuple(accs)

            zero = jnp.zeros((16,), jnp.float32)
            accs = lax.fori_loop(0, 4, cbody, tuple(zero for _ in range(16)))
            for m in range(M_):
                out_v[b, m, 0:16] = accs[2 * m]
                out_v[b, m, 16:32] = accs[2 * m + 1]
            pltpu.async_copy(out_v.at[b], out_hbm.at[g], sem_o[b])

        def step(g, b, nb, has_next, has_next2, has_prev_out):
            if has_next:
                wait_idx(g + 1, nb)
                fire_gathers(nb)
            wait_gathers(b)
            if has_next2:
                load_idx(g + 2, b)
            if has_prev_out:
                wait_out(b)
            wait_w(g, b)
            compute_store(g, b)
            if has_next2:
                load_w(g + 2, b)

        load_idx(qbase, 0)
        load_w(qbase, 0)
        wait_idx(qbase, 0)
        fire_gathers(0)
        load_idx(qbase + 1, 1)
        load_w(qbase + 1, 1)
        step(qbase, 0, 1, True, True, False)
        step(qbase + 1, 1, 0, True, True, False)

        def pair_body(t, carry):
            g = qbase + 2 + 2 * t
            step(g, 0, 1, True, True, True)
            step(g + 1, 1, 0, True, True, True)
            return carry

        lax.fori_loop(0, (qpw - 4) // 2, pair_body, 0)
        step(qbase + qpw - 2, 0, 1, True, False, True)
        step(qbase + qpw - 1, 1, 0, False, False, True)
        wait_out(0)
        wait_out(1)

    return k(table, idxs, wgts)


def kernel(query, reference_points, input_flatten, input_spatial_shapes,
           input_level_start_index, Wv, bv, Woff, boff, Wattn, battn, Wout,
           bout):
    q2 = query.reshape(LQ_, C_)

    perm = np.array(
        [m * 32 + (p % 2) * 16 + p // 2 for m in range(M_) for p in range(32)])
    value_bf = _matbias(input_flatten.reshape(LEN_IN_, C_), Wv[:, perm],
                        bv[perm], _BV, out_dtype=jnp.bfloat16)
    table = value_bf.reshape(NROWS_, D_)

    woff_r = Woff.reshape(C_, 128, 3).transpose(0, 2, 1).reshape(C_, 384)
    boff_r = boff.reshape(128, 3).transpose(1, 0).reshape(1, 384)
    rp_t = jnp.transpose(reference_points.reshape(LQ_, L_, 3), (0, 2, 1))
    rp_lane = jnp.tile(jnp.repeat(rp_t, P_, axis=2), (1, 1, M_))

    idxs, wgts = _sample_prep(q2, woff_r, boff_r, Wattn, battn.reshape(1, 128),
                              rp_lane)

    sc_out = _sc_sample(table, idxs, wgts)

    out = _matbias(sc_out.reshape(LQ_, C_), Wout, bout, _BQ)
    return out.reshape(1, LQ_, C_)


# speedup vs baseline: 1.0039x; 1.0039x over previous
import functools

import jax
import jax.numpy as jnp
import numpy as np
from jax import lax
from jax.experimental import pallas as pl
from jax.experimental.pallas import tpu as pltpu
from jax.experimental.pallas import tpu_sc as plsc

M_ = 8
L_ = 4
P_ = 4
D_ = 32
C_ = 256
LQ_ = 4096
LEN_IN_ = 43520
NROWS_ = LEN_IN_ * M_

_BQ = 512
_BV = 512


def _matbias_body(x_ref, w_ref, b_ref, o_ref):
    o_ref[...] = (
        jnp.dot(x_ref[...], w_ref[...], preferred_element_type=jnp.float32)
        + b_ref[...]
    ).astype(o_ref.dtype)


def _matbias(x, w, b, bm, out_dtype=jnp.float32):
    n, k = x.shape
    ko = w.shape[1]
    return pl.pallas_call(
        _matbias_body,
        grid=(n // bm,),
        in_specs=[
            pl.BlockSpec((bm, k), lambda i: (i, 0)),
            pl.BlockSpec((k, ko), lambda i: (0, 0)),
            pl.BlockSpec((1, ko), lambda i: (0, 0)),
        ],
        out_specs=pl.BlockSpec((bm, ko), lambda i: (i, 0)),
        out_shape=jax.ShapeDtypeStruct((n, ko), out_dtype),
    )(x, w, b.reshape(1, ko))


def _sample_prep_body(q_ref, woff_ref, boff_ref, wattn_ref, battn_ref, rp_ref,
                      idx_ref, w_ref):
    q = q_ref[...]
    off = (
        jnp.dot(q, woff_ref[...], preferred_element_type=jnp.float32)
        + boff_ref[...]
    )
    logits = (
        jnp.dot(q, wattn_ref[...], preferred_element_type=jnp.float32)
        + battn_ref[...]
    )
    a3 = logits.reshape(_BQ, M_, L_ * P_)
    a3 = a3 - jnp.max(a3, axis=-1, keepdims=True)
    e3 = jnp.exp(a3)
    aw = (e3 / jnp.sum(e3, axis=-1, keepdims=True)).reshape(_BQ, 128)

    j = lax.broadcasted_iota(jnp.int32, (1, 128), 1)
    m_i = j // 16
    lvl = (j % 16) // 4
    wi = jnp.right_shift(64, lvl)
    hi = wi
    ti = 8
    start_i = jnp.where(
        lvl == 0, 0,
        jnp.where(lvl == 1, 32768, jnp.where(lvl == 2, 40960, 43008)))
    wf = wi.astype(jnp.float32)
    hf = wf
    tf = 8.0

    ix = (rp_ref[:, 0, :] + off[:, 0:128] / tf) * wf - 0.5
    iy = (rp_ref[:, 1, :] + off[:, 128:256] / wf) * hf - 0.5
    iz = (rp_ref[:, 2, :] + off[:, 256:384] / hf) * tf - 0.5

    x0f = jnp.floor(ix)
    y0f = jnp.floor(iy)
    z0f = jnp.floor(iz)
    fx = ix - x0f
    fy = iy - y0f
    fz = iz - z0f
    x0 = x0f.astype(jnp.int32)
    y0 = y0f.astype(jnp.int32)
    z0 = z0f.astype(jnp.int32)

    vx = []
    vy = []
    vz = []
    xcol = []
    yrow = []
    zrow = []
    base = start_i * M_ + m_i
    for d in (0, 1):
        xi = x0 + d
        yi = y0 + d
        zi = z0 + d
        vx.append((xi >= 0) & (xi < wi))
        vy.append((yi >= 0) & (yi < hi))
        vz.append((zi >= 0) & (zi < ti))
        xcol.append(jnp.clip(xi, 0, wi - 1) * M_)
        yrow.append(jnp.clip(yi, 0, hi - 1) * (wi * M_))
        zrow.append(jnp.clip(zi, 0, ti - 1) * hi * (wi * M_) + base)
    wxa = (1.0 - fx, fx)
    wya = (1.0 - fy, fy)
    wza = (aw * (1.0 - fz), aw * fz)
    for dy in (0, 1):
        for dx in (0, 1):
            vxy = vx[dx] & vy[dy]
            wxy = wxa[dx] * wya[dy]
            sxy = yrow[dy] + xcol[dx]
            for dz in (0, 1):
                c = dz * 4 + dy * 2 + dx
                row = zrow[dz] + sxy
                wgt = jnp.where(vxy & vz[dz], wxy * wza[dz], 0.0)
                idx_ref[:, c, :] = row
                w_ref[:, c, :] = wgt


def _sample_prep(query, woff_r, boff_r, wattn, battn, rp_lane):
    return pl.pallas_call(
        _sample_prep_body,
        grid=(LQ_ // _BQ,),
        in_specs=[
            pl.BlockSpec((_BQ, C_), lambda i: (i, 0)),
            pl.BlockSpec((C_, 3 * 128), lambda i: (0, 0)),
            pl.BlockSpec((1, 3 * 128), lambda i: (0, 0)),
            pl.BlockSpec((C_, 128), lambda i: (0, 0)),
            pl.BlockSpec((1, 128), lambda i: (0, 0)),
            pl.BlockSpec((_BQ, 3, 128), lambda i: (i, 0, 0)),
        ],
        out_specs=[
            pl.BlockSpec((_BQ, 8, 128), lambda i: (i, 0, 0)),
            pl.BlockSpec((_BQ, 8, 128), lambda i: (i, 0, 0)),
        ],
        out_shape=[
            jax.ShapeDtypeStruct((LQ_, 8, 128), jnp.int32),
            jax.ShapeDtypeStruct((LQ_, 8, 128), jnp.float32),
        ],
    )(query, woff_r, boff_r, wattn, battn, rp_lane)


def _sc_sample(table, idxs, wgts):
    info = plsc.get_sparse_core_info()
    nc, ns = info.num_cores, info.num_subcores
    nw = nc * ns
    qpw = LQ_ // nw
    mesh = plsc.VectorSubcoreMesh(core_axis_name="c", subcore_axis_name="s")

    @functools.partial(
        pl.kernel,
        out_type=jax.ShapeDtypeStruct((LQ_, M_, D_), jnp.float32),
        mesh=mesh,
        compiler_params=pltpu.CompilerParams(
            use_tc_tiling_on_sc=False, needs_layout_passes=False),
        scratch_types=[
            pltpu.VMEM((2, 8, 128), jnp.int32),
            pltpu.VMEM((2, 8, 128), jnp.float32),
            pltpu.VMEM((2, 8, 128, D_), jnp.bfloat16),
            pltpu.VMEM((2, M_, D_), jnp.float32),
            pltpu.SemaphoreType.DMA,
            pltpu.SemaphoreType.DMA,
            pltpu.SemaphoreType.DMA,
            pltpu.SemaphoreType.DMA,
            pltpu.SemaphoreType.DMA,
            pltpu.SemaphoreType.DMA,
            pltpu.SemaphoreType.DMA,
            pltpu.SemaphoreType.DMA,
        ],
    )
    def k(table_hbm, idx_hbm, w_hbm, out_hbm, idx_v, w_v, rows_v, out_v,
          si0, si1, sw0, sw1, sg0, sg1, so0, so1):
        wid = lax.axis_index("s") * nc + lax.axis_index("c")
        qbase = wid * qpw
        sem_i = (si0, si1)
        sem_w = (sw0, sw1)
        sem_g = (sg0, sg1)
        sem_o = (so0, so1)

        def load_idx(g, b):
            pltpu.async_copy(idx_hbm.at[g], idx_v.at[b], sem_i[b])

        def wait_idx(g, b):
            pltpu.make_async_copy(idx_hbm.at[g], idx_v.at[b], sem_i[b]).wait()

        def load_w(g, b):
            pltpu.async_copy(w_hbm.at[g], w_v.at[b], sem_w[b])

        def wait_w(g, b):
            pltpu.make_async_copy(w_hbm.at[g], w_v.at[b], sem_w[b]).wait()

        def fire_gathers(b):
            for c in range(8):
                pltpu.async_copy(
                    table_hbm.at[idx_v.at[b, c]], rows_v.at[b, c], sem_g[b])

        def wait_gathers(b):
            for c in range(8):
                pltpu.make_async_copy(
                    table_hbm.at[idx_v.at[b, c]], rows_v.at[b, c],
                    sem_g[b]).wait()

        def wait_out(b):
            pltpu.make_async_copy(
                out_v.at[b], out_hbm.at[qbase], sem_o[b]).wait()

        def compute_store(g, b):
            def cbody(c, accs):
                accs = list(accs)
                for m in range(M_):
                    a0 = accs[2 * m]
                    a1 = accs[2 * m + 1]
                    wvec = w_v[b, c, pl.ds(m * 16, 16)]
                    for kk in range(16):
                        j = m * 16 + kk
                        wv = wvec[kk]
                        lo, hi = plsc.unpack(
                            rows_v[b, c, j, 0:32],
                            format=plsc.PackFormat.INTERLEAVED)
                        a0 = a0 + wv * lo
                        a1 = a1 + wv * hi
                    accs[2 * m] = a0
                    accs[2 * m + 1] = a1
                return tuple(accs)

            zero = jnp.zeros((16,), jnp.float32)
            accs = lax.fori_loop(0, 8, cbody, tuple(zero for _ in range(16)))
            for m in range(M_):
                out_v[b, m, 0:16] = accs[2 * m]
                out_v[b, m, 16:32] = accs[2 * m + 1]
            pltpu.async_copy(out_v.at[b], out_hbm.at[g], sem_o[b])

        def step(g, b, nb, has_next, has_next2, has_prev_out):
            if has_next:
                wait_idx(g + 1, nb)
                fire_gathers(nb)
            wait_gathers(b)
            if has_next2:
                load_idx(g + 2, b)
            if has_prev_out:
                wait_out(b)
            wait_w(g, b)
            compute_store(g, b)
            if has_next2:
                load_w(g + 2, b)

        load_idx(qbase, 0)
        load_w(qbase, 0)
        wait_idx(qbase, 0)
        fire_gathers(0)
        load_idx(qbase + 1, 1)
        load_w(qbase + 1, 1)
        step(qbase, 0, 1, True, True, False)
        step(qbase + 1, 1, 0, True, True, False)

        def pair_body(t, carry):
            g = qbase + 2 + 2 * t
            step(g, 0, 1, True, True, True)
            step(g + 1, 1, 0, True, True, True)
            return carry

        lax.fori_loop(0, (qpw - 4) // 2, pair_body, 0)
        step(qbase + qpw - 2, 0, 1, True, False, True)
        step(qbase + qpw - 1, 1, 0, False, False, True)
        wait_out(0)
        wait_out(1)

    return k(table, idxs, wgts)


def kernel(query, reference_points, input_flatten, input_spatial_shapes,
           input_level_start_index, Wv, bv, Woff, boff, Wattn, battn, Wout,
           bout):
    q2 = query.reshape(LQ_, C_)

    perm = np.array(
        [m * 32 + (p % 2) * 16 + p // 2 for m in range(M_) for p in range(32)])
    value_bf = _matbias(input_flatten.reshape(LEN_IN_, C_), Wv[:, perm],
                        bv[perm], _BV, out_dtype=jnp.bfloat16)
    table = value_bf.reshape(NROWS_, D_)

    woff_r = Woff.reshape(C_, 128, 3).transpose(0, 2, 1).reshape(C_, 384)
    boff_r = boff.reshape(128, 3).transpose(1, 0).reshape(1, 384)
    rp_t = jnp.transpose(reference_points.reshape(LQ_, L_, 3), (0, 2, 1))
    rp_lane = jnp.tile(jnp.repeat(rp_t, P_, axis=2), (1, 1, M_))

    idxs, wgts = _sample_prep(q2, woff_r, boff_r, Wattn, battn.reshape(1, 128),
                              rp_lane)

    sc_out = _sc_sample(table, idxs, wgts)

    out = _matbias(sc_out.reshape(LQ_, C_), Wout, bout, _BQ)
    return out.reshape(1, LQ_, C_)
